# Initial kernel scaffold; baseline (speedup 1.0000x reference)
import jax
import jax.numpy as jnp
from jax.experimental import pallas as pl


def kernel(x, weight):
    raise NotImplementedError("write your pallas kernel here")


# speedup vs baseline: 5.7786x; 5.7786x over previous
import functools

import jax
import jax.numpy as jnp
from jax import lax
from jax.experimental import pallas as pl
from jax.experimental.pallas import tpu as pltpu
from jax.experimental.pallas import tpu_sc as plsc

NUM_EMB = 100000
DIM = 128
BATCH = 4096
HIST = 200
TOT = BATCH * HIST
NC, NS = 2, 16
NW = NC * NS
PER_W = TOT // NW
CHUNK = 128
NCHUNK = PER_W // CHUNK


def _sc_gather(x_flat, weight):
    mesh = plsc.VectorSubcoreMesh(core_axis_name="c", subcore_axis_name="s")

    @functools.partial(
        pl.kernel,
        out_type=jax.ShapeDtypeStruct((TOT, DIM), jnp.float32),
        mesh=mesh,
        scratch_types=[
            pltpu.VMEM((NCHUNK, CHUNK), jnp.int32),
            pltpu.VMEM((CHUNK, DIM), jnp.float32),
            pltpu.SemaphoreType.DMA,
        ],
    )
    def k(x_hbm, table_hbm, out_hbm, idx_v, rows_v, sem):
        wid = lax.axis_index("s") * NC + lax.axis_index("c")
        base = wid * PER_W
        pltpu.sync_copy(x_hbm.at[wid], idx_v)

        def body(j, _):
            pltpu.async_copy(table_hbm.at[idx_v.at[j]], rows_v, sem).wait()
            off = pl.multiple_of(base + j * CHUNK, CHUNK)
            pltpu.sync_copy(rows_v, out_hbm.at[pl.ds(off, CHUNK)])
            return _

        lax.fori_loop(0, NCHUNK, body, None)

    return k(x_flat, weight)


def kernel(x, weight):
    x_flat = x.reshape(NW, NCHUNK, CHUNK)
    out = _sc_gather(x_flat, weight)
    return out.reshape(BATCH, HIST, DIM)


# speedup vs baseline: 8.4377x; 1.4602x over previous
import functools

import jax
import jax.numpy as jnp
from jax import lax
from jax.experimental import pallas as pl
from jax.experimental.pallas import tpu as pltpu
from jax.experimental.pallas import tpu_sc as plsc

NUM_EMB = 100000
DIM = 128
BATCH = 4096
HIST = 200
TOT = BATCH * HIST
NC, NS = 2, 16
NW = NC * NS
PER_W = TOT // NW
CHUNK = 128
NCHUNK = PER_W // CHUNK
NBUF = 4
LEAD = 2


def _sc_gather(x_flat, weight):
    mesh = plsc.VectorSubcoreMesh(core_axis_name="c", subcore_axis_name="s")

    @functools.partial(
        pl.kernel,
        out_type=jax.ShapeDtypeStruct((TOT, DIM), jnp.float32),
        mesh=mesh,
        scratch_types=[
            pltpu.VMEM((NCHUNK, CHUNK), jnp.int32),
            [pltpu.VMEM((CHUNK, DIM), jnp.float32) for _ in range(NBUF)],
            [pltpu.SemaphoreType.DMA for _ in range(NBUF)],
            [pltpu.SemaphoreType.DMA for _ in range(NBUF)],
        ],
    )
    def k(x_hbm, table_hbm, out_hbm, idx_v, rows, sem_g, sem_p):
        wid = lax.axis_index("s") * NC + lax.axis_index("c")
        base = wid * PER_W
        pltpu.sync_copy(x_hbm.at[wid], idx_v)

        def gather_start(j, b):
            pltpu.make_async_copy(
                table_hbm.at[idx_v.at[j]], rows[b], sem_g[b]).start()

        def gather_wait(j, b):
            pltpu.make_async_copy(
                table_hbm.at[idx_v.at[j]], rows[b], sem_g[b]).wait()

        def put_descr(j, b):
            off = pl.multiple_of(base + j * CHUNK, CHUNK)
            return pltpu.make_async_copy(
                rows[b], out_hbm.at[pl.ds(off, CHUNK)], sem_p[b])

        for b in range(LEAD):
            gather_start(b, b)

        def body(g, _):
            for b in range(NBUF):
                j = g * NBUF + b
                bn = (b + LEAD) % NBUF
                jn = j + LEAD

                @pl.when(jn < NCHUNK)
                def _():
                    @pl.when(j >= LEAD)
                    def _():
                        put_descr(j - LEAD, bn).wait()
                    gather_start(jn, bn)

                gather_wait(j, b)
                put_descr(j, b).start()
            return _

        lax.fori_loop(0, NCHUNK // NBUF, body, None)

        for b in range(NBUF):
            put_descr(NCHUNK - NBUF + b, b).wait()

    return k(x_flat, weight)


def kernel(x, weight):
    x_flat = x.reshape(NW, NCHUNK, CHUNK)
    out = _sc_gather(x_flat, weight)
    return out.reshape(BATCH, HIST, DIM)
